# baseline (speedup 1.0000x reference)
import jax
import jax.numpy as jnp
from jax import lax
from jax.experimental import pallas as pl
from jax.experimental.pallas import tpu as pltpu
from jax.experimental.pallas import tpu_sc as plsc

B = 1024
L = 50
D = 64
NR = 5
NW = 32
NCHUNK = 1
CB = B // NCHUNK
BIG = CB * L
BPW = BIG // NW
SPW = CB // NW
BB = 64
G = CB // BB

_f32 = jnp.float32


def _sc_gather(hist_v, soc, hist_u, nodes_u, nodes_v, embed_u, embed_i):
  mesh = plsc.VectorSubcoreMesh(core_axis_name="c", subcore_axis_name="s")
  out_type = (
      jax.ShapeDtypeStruct((BIG, D), _f32),
      jax.ShapeDtypeStruct((BIG, D), _f32),
      jax.ShapeDtypeStruct((BIG, D), _f32),
      jax.ShapeDtypeStruct((CB, D), _f32),
      jax.ShapeDtypeStruct((CB, D), _f32),
  )

  def body(hist_v, soc, hist_u, nodes_u, nodes_v, embed_u, embed_i,
           pt_out, un_out, qa_out, piu_out, qj_out,
           idx_v, rows_v, idx_s, rows_s, sem):
    wid = lax.axis_index("s") * 2 + lax.axis_index("c")
    base = wid * BPW
    for idx_hbm, table, out in ((hist_v, embed_u, pt_out),
                                (soc, embed_u, un_out),
                                (hist_u, embed_i, qa_out)):
      pltpu.sync_copy(idx_hbm.at[pl.ds(base, BPW)], idx_v)
      pltpu.async_copy(table.at[idx_v], rows_v, sem).wait()
      pltpu.sync_copy(rows_v, out.at[pl.ds(base, BPW)])
    sbase = wid * SPW
    for idx_hbm, table, out in ((nodes_u, embed_u, piu_out),
                                (nodes_v, embed_i, qj_out)):
      pltpu.sync_copy(idx_hbm.at[pl.ds(sbase, SPW)], idx_s)
      pltpu.async_copy(table.at[idx_s], rows_s, sem).wait()
      pltpu.sync_copy(rows_s, out.at[pl.ds(sbase, SPW)])

  return pl.kernel(
      body,
      out_type=out_type,
      mesh=mesh,
      scratch_types=[
          pltpu.VMEM((BPW,), jnp.int32),
          pltpu.VMEM((BPW, D), _f32),
          pltpu.VMEM((SPW,), jnp.int32),
          pltpu.VMEM((SPW, D), _f32),
          pltpu.SemaphoreType.DMA,
      ],
      compiler_params=pltpu.CompilerParams(use_tc_tiling_on_sc=False),
  )(hist_v, soc, hist_u, nodes_u, nodes_v, embed_u, embed_i)


def _tc_body(pt_ref, qa_ref, un_ref, qj_ref, piu_ref, vr_ref, ur_ref,
             erp2, w1, w1r, b1, w2, b2,
             a1, a1u, b1a, a2, b2a, a3two,
             aS1a, aS1b, aS1bias, aS2w, aS2b, a3twoS,
             sel_ref, selt_ref, qab_ref, onesab_ref,
             um1a, um1b, um1bias, um2w, um2b,
             ur1w, ur1b, ur2w, ur2b,
             vr1w, vr1b, vr2w, vr2b,
             uv1a, uv1b, uv1bias, uv2w, uv2b, uv3wT, uv3b,
             s1, t1, s2, t2, s3, t3, s4, t4,
             out_ref):
  def mm(x, w):
    return lax.dot_general(x, w, (((1,), (0,)), ((), ())),
                           preferred_element_type=_f32)

  def mmh(x, w):
    return lax.dot_general(x, w, (((1,), (0,)), ((), ())),
                           preferred_element_type=_f32,
                           precision=lax.Precision.HIGHEST)

  relu = lambda x: jnp.maximum(x, 0.0)

  def softmax_l(logits):
    m = jnp.max(logits, axis=1, keepdims=True)
    e = jnp.exp(logits - m)
    return e / jnp.sum(e, axis=1, keepdims=True)

  sel = sel_ref[...]
  selt = selt_ref[...]
  qab = qab_ref[...]
  onesab = onesab_ref[...]

  def att_pair(Fm, Um, a1_, a1u_, b1a_, a2_, a3two_, b2a_=None):
    HU = mm(Um, a1u_) + b1a_
    H3 = relu(mm(Fm, a1_) + mmh(sel, HU))
    H2 = mm(H3, a2_)
    if b2a_ is not None:
      H2 = H2 + b2a_
    H2 = relu(H2)
    LGr = mm(H2, a3two_)
    LGc = mmh(selt, LGr * qab)
    wA = softmax_l(LGc[:, :L])
    wB = softmax_l(LGc[:, L:2 * L])
    Wc = jnp.concatenate(
        [wA, wB, jnp.zeros((BB, 128 - 2 * L), _f32)], axis=1)
    MUr = mmh(mmh(sel, Wc) * qab, onesab)
    return mmh(selt, Fm * MUr[:, :Fm.shape[1]])

  X = jnp.concatenate([pt_ref[...], qa_ref[...]], axis=1)
  U = jnp.concatenate([qj_ref[...], piu_ref[...]], axis=1)
  m16 = lax.broadcasted_iota(jnp.int32, (BB * L, 16), 1)
  idsel = jnp.where(m16 < 8, vr_ref[...], ur_ref[...])
  oh = (idsel == (m16 & 7)).astype(_f32)
  ER = mm(oh, erp2[...])
  H = relu(mm(X, w1[...]) + mm(ER, w1r[...]) + b1[...])
  F = relu(mm(H, w2[...]) + b2[...])

  pool = att_pair(F, U, a1[...], a1u[...], b1a[...], a2[...], a3two[...],
                  b2a[...])
  zj = pool[:, :D]
  hi_I = pool[:, D:]

  un = un_ref[...]
  piu = piu_ref[...]
  hi_S = att_pair(un, piu, aS1a[...], aS1b[...], aS1bias[...], aS2w[...],
                  a3twoS[...], aS2b[...])[:, :D]
  hi = relu(mm(hi_I, um1a[...]) + mm(hi_S, um1b[...]) + um1bias[...])
  hi = relu(mm(hi, um2w[...]) + um2b[...])
  hi = relu((mm(hi, ur1w[...]) + ur1b[...]) * s1[...] + t1[...])
  hi = mm(hi, ur2w[...]) + ur2b[...]
  zj = relu((mm(zj, vr1w[...]) + vr1b[...]) * s2[...] + t2[...])
  zj = relu(mm(zj, vr1w[...]) + vr1b[...])
  zj = mm(zj, vr2w[...]) + vr2b[...]
  x = relu((mm(hi, uv1a[...]) + mm(zj, uv1b[...]) + uv1bias[...]) * s3[...]
           + t3[...])
  x = relu((mm(x, uv2w[...]) + uv2b[...]) * s4[...] + t4[...])
  out_ref[...] = (jnp.sum(x * uv3wT[...], axis=1, keepdims=True) + uv3b[...])


def kernel(nodes_u, nodes_v, history_u, history_ur, history_v, history_vr,
           social_adj, embed_u, embed_i, embed_r,
           gu1_w, gu1_b, gu2_w, gu2_b, ai1_w, ai1_b, ai2_w, ai2_b, ai3_w,
           ai3_b, gv1_w, gv1_b, gv2_w, gv2_b, aI1_w, aI1_b, aI2_w, aI2_b,
           aI3_w, aI3_b, aS1_w, aS1_b, aS2_w, aS2_b, aS3_w, aS3_b, um1_w,
           um1_b, um2_w, um2_b, ur1_w, ur1_b, ur2_w, ur2_b, vr1_w, vr1_b,
           vr2_w, vr2_b, uv1_w, uv1_b, uv2_w, uv2_b, uv3_w, uv3_b,
           bn1_g, bn1_b, bn1_m, bn1_v, bn2_g, bn2_b, bn2_m, bn2_v,
           bn3_g, bn3_b, bn3_m, bn3_v, bn4_g, bn4_b, bn4_m, bn4_v):
  i32 = jnp.int32
  hist_v = history_v.astype(i32).reshape(B * L)
  soc = social_adj.astype(i32).reshape(B * L)
  hist_u = history_u.astype(i32).reshape(B * L)
  nu = nodes_u.astype(i32)
  nv = nodes_v.astype(i32)
  vr_ids = history_vr.astype(i32).reshape(B * L, 1)
  ur_ids = history_ur.astype(i32).reshape(B * L, 1)

  erp = jnp.pad(embed_r, ((0, 8 - NR), (0, 0)))

  def bn_fold(g, b, m, v):
    s = (g * lax.rsqrt(v + 1e-5)).reshape(1, -1)
    t = (b - m * g * lax.rsqrt(v + 1e-5)).reshape(1, -1)
    return s, t

  s1, t1 = bn_fold(bn1_g, bn1_b, bn1_m, bn1_v)
  s2, t2 = bn_fold(bn2_g, bn2_b, bn2_m, bn2_v)
  s3, t3 = bn_fold(bn3_g, bn3_b, bn3_m, bn3_v)
  s4, t4 = bn_fold(bn4_g, bn4_b, bn4_m, bn4_v)

  row = lambda b: b.reshape(1, -1)

  def bdiag(a, b):
    za = jnp.zeros_like(a)
    return jnp.concatenate(
        [jnp.concatenate([a, za], axis=1), jnp.concatenate([za, b], axis=1)],
        axis=0)

  ii = jnp.arange(BB * L)
  cc = jnp.arange(128)
  rr = jnp.arange(BB)
  sel = (ii[:, None] // L == rr[None, :]).astype(_f32)
  selt = (rr[:, None] == ii[None, :] // L).astype(_f32)
  qab = ((ii[:, None] % L == cc[None, :])
         | (ii[:, None] % L + L == cc[None, :])).astype(_f32)
  onesab = (((cc[:, None] < L) & (cc[None, :] < D))
            | ((cc[:, None] >= L) & (cc[:, None] < 2 * L)
               & (cc[None, :] >= D))).astype(_f32)
  a3two = jnp.concatenate(
      [jnp.where(cc[None, :] < L, ai3_w, 0.0),
       jnp.where((cc[None, :] >= L) & (cc[None, :] < 2 * L), aI3_w, 0.0)],
      axis=0)
  a3twoS = jnp.where(cc[None, :] < L, aS3_w, 0.0)

  weights = [
      bdiag(erp, erp),
      bdiag(gu1_w[:D], gv1_w[:D]), bdiag(gu1_w[D:], gv1_w[D:]),
      row(jnp.concatenate([gu1_b, gv1_b])),
      bdiag(gu2_w, gv2_w), row(jnp.concatenate([gu2_b, gv2_b])),
      bdiag(ai1_w[:D], aI1_w[:D]), bdiag(ai1_w[D:], aI1_w[D:]),
      row(jnp.concatenate([ai1_b, aI1_b])),
      bdiag(ai2_w, aI2_w), row(jnp.concatenate([ai2_b, aI2_b])),
      a3two,
      aS1_w[:D], aS1_w[D:], row(aS1_b), aS2_w, row(aS2_b), a3twoS,
      sel, selt, qab, onesab,
      um1_w[:D], um1_w[D:], row(um1_b), um2_w, row(um2_b),
      ur1_w, row(ur1_b), ur2_w, row(ur2_b),
      vr1_w, row(vr1_b), vr2_w, row(vr2_b),
      uv1_w[:D], uv1_w[D:], row(uv1_b), uv2_w, row(uv2_b),
      uv3_w.reshape(1, 16), uv3_b.reshape(1, 1),
      s1, t1, s2, t2, s3, t3, s4, t4,
  ]

  big_spec = pl.BlockSpec((BB * L, D), lambda g: (g, 0))
  small_spec = pl.BlockSpec((BB, D), lambda g: (g, 0))
  ids_spec = pl.BlockSpec((BB * L, 1), lambda g: (g, 0))
  wspec = lambda a: pl.BlockSpec(a.shape, lambda g, _n=len(a.shape): (0,) * _n)

  in_specs = ([big_spec, big_spec, big_spec, small_spec, small_spec,
               ids_spec, ids_spec]
              + [wspec(w) for w in weights])

  tc_call = pl.pallas_call(
      _tc_body,
      grid=(G,),
      in_specs=in_specs,
      out_specs=pl.BlockSpec((BB, 1), lambda g: (g, 0)),
      out_shape=jax.ShapeDtypeStruct((CB, 1), _f32),
      compiler_params=pltpu.CompilerParams(
          dimension_semantics=("arbitrary",),
          vmem_limit_bytes=56 * 1024 * 1024),
  )

  outs = []
  for c in range(NCHUNK):
    lo, slo = c * BIG, c * CB
    pt, un, qa, piu, qj = _sc_gather(
        lax.slice(hist_v, (lo,), (lo + BIG,)),
        lax.slice(soc, (lo,), (lo + BIG,)),
        lax.slice(hist_u, (lo,), (lo + BIG,)),
        lax.slice(nu, (slo,), (slo + CB,)),
        lax.slice(nv, (slo,), (slo + CB,)),
        embed_u, embed_i)
    vr_c = lax.slice(vr_ids, (lo, 0), (lo + BIG, 1))
    ur_c = lax.slice(ur_ids, (lo, 0), (lo + BIG, 1))
    outs.append(tc_call(pt, qa, un, qj, piu, vr_c, ur_c, *weights))

  return jnp.concatenate(outs, axis=0)[:, 0]


# speedup vs baseline: 1.4841x; 1.4841x over previous
import jax
import jax.numpy as jnp
from jax import lax
from jax.experimental import pallas as pl
from jax.experimental.pallas import tpu as pltpu
from jax.experimental.pallas import tpu_sc as plsc

B = 1024
L = 50
D = 64
NR = 5
NW = 32
NCHUNK = 1
CB = B // NCHUNK
BIG = CB * L
BPW = BIG // NW
SPW = CB // NW
BB = 128
G = CB // BB

_f32 = jnp.float32


def _sc_gather(hist_v, soc, hist_u, nodes_u, nodes_v, embed_u, embed_i):
  mesh = plsc.VectorSubcoreMesh(core_axis_name="c", subcore_axis_name="s")
  out_type = (
      jax.ShapeDtypeStruct((BIG, D), _f32),
      jax.ShapeDtypeStruct((BIG, D), _f32),
      jax.ShapeDtypeStruct((BIG, D), _f32),
      jax.ShapeDtypeStruct((CB, D), _f32),
      jax.ShapeDtypeStruct((CB, D), _f32),
  )

  def body(hist_v, soc, hist_u, nodes_u, nodes_v, embed_u, embed_i,
           pt_out, un_out, qa_out, piu_out, qj_out,
           idx_v, rows_v, idx_s, rows_s, sem):
    wid = lax.axis_index("s") * 2 + lax.axis_index("c")
    base = wid * BPW
    for idx_hbm, table, out in ((hist_v, embed_u, pt_out),
                                (soc, embed_u, un_out),
                                (hist_u, embed_i, qa_out)):
      pltpu.sync_copy(idx_hbm.at[pl.ds(base, BPW)], idx_v)
      pltpu.async_copy(table.at[idx_v], rows_v, sem).wait()
      pltpu.sync_copy(rows_v, out.at[pl.ds(base, BPW)])
    sbase = wid * SPW
    for idx_hbm, table, out in ((nodes_u, embed_u, piu_out),
                                (nodes_v, embed_i, qj_out)):
      pltpu.sync_copy(idx_hbm.at[pl.ds(sbase, SPW)], idx_s)
      pltpu.async_copy(table.at[idx_s], rows_s, sem).wait()
      pltpu.sync_copy(rows_s, out.at[pl.ds(sbase, SPW)])

  return pl.kernel(
      body,
      out_type=out_type,
      mesh=mesh,
      scratch_types=[
          pltpu.VMEM((BPW,), jnp.int32),
          pltpu.VMEM((BPW, D), _f32),
          pltpu.VMEM((SPW,), jnp.int32),
          pltpu.VMEM((SPW, D), _f32),
          pltpu.SemaphoreType.DMA,
      ],
      compiler_params=pltpu.CompilerParams(use_tc_tiling_on_sc=False),
  )(hist_v, soc, hist_u, nodes_u, nodes_v, embed_u, embed_i)


def _tc_body(pt_ref, qa_ref, un_ref, qj_ref, piu_ref, vr_ref, ur_ref,
             erp2, w1, w1r, b1, w2, b2,
             a1, a1u, b1a, a2, b2a, a3c,
             aS1a, aS1b, aS1bias, aS2w, aS2b, aS3wT,
             um1a, um1b, um1bias, um2w, um2b,
             ur1w, ur1b, ur2w, ur2b,
             vr1w, vr1b, vr2w, vr2b,
             uv1a, uv1b, uv1bias, uv2w, uv2b, uv3wT, uv3b,
             s1, t1, s2, t2, s3, t3, s4, t4,
             out_ref):
  def mm(x, w):
    return lax.dot_general(x, w, (((1,), (0,)), ((), ())),
                           preferred_element_type=_f32)

  relu = lambda x: jnp.maximum(x, 0.0)
  bf = lambda x: x.astype(jnp.bfloat16).astype(_f32)

  def softmax_l(logits):
    m = jnp.max(logits, axis=1, keepdims=True)
    e = jnp.exp(logits - m)
    return e / jnp.sum(e, axis=1, keepdims=True)

  X = jnp.concatenate([pt_ref[...], qa_ref[...]], axis=1)
  U = jnp.concatenate([qj_ref[...], piu_ref[...]], axis=1)
  m16 = lax.broadcasted_iota(jnp.int32, (BB * L, 16), 1)
  idsel = jnp.where(m16 < 8, vr_ref[...], ur_ref[...])
  oh = (idsel == (m16 & 7)).astype(_f32)
  ER = mm(oh, erp2[...])
  H = relu(mm(X, w1[...]) + mm(ER, w1r[...]) + b1[...])
  F = relu(mm(H, w2[...]) + b2[...])
  HA = mm(F, a1[...])
  HU = mm(U, a1u[...]) + b1a[...]
  H3 = relu(HA.reshape(BB, L, 128) + HU[:, None, :])
  H2 = relu(mm(H3.reshape(BB * L, 128), a2[...]) + b2a[...])
  T3 = (bf(H2) * bf(a3c[...])).reshape(BB, L, 128)
  wA = softmax_l(jnp.sum(T3[:, :, :D], axis=2))
  wB = softmax_l(jnp.sum(T3[:, :, D:], axis=2))
  F3 = F.reshape(BB, L, 128)
  zj = jnp.sum(F3[:, :, :D] * wA[:, :, None], axis=1)
  hi_I = jnp.sum(F3[:, :, D:] * wB[:, :, None], axis=1)

  un = un_ref[...]
  piu = piu_ref[...]
  h = mm(un, aS1a[...])
  hu = mm(piu, aS1b[...]) + aS1bias[...]
  h3 = relu(h.reshape(BB, L, D) + hu[:, None, :])
  h2 = relu(mm(h3.reshape(BB * L, D), aS2w[...]) + aS2b[...])
  tS = (bf(h2) * bf(aS3wT[...])).reshape(BB, L, D)
  wS = softmax_l(jnp.sum(tS, axis=2))
  hi_S = jnp.sum(un.reshape(BB, L, D) * wS[:, :, None], axis=1)
  hi = relu(mm(hi_I, um1a[...]) + mm(hi_S, um1b[...]) + um1bias[...])
  hi = relu(mm(hi, um2w[...]) + um2b[...])
  hi = relu((mm(hi, ur1w[...]) + ur1b[...]) * s1[...] + t1[...])
  hi = mm(hi, ur2w[...]) + ur2b[...]
  zj = relu((mm(zj, vr1w[...]) + vr1b[...]) * s2[...] + t2[...])
  zj = relu(mm(zj, vr1w[...]) + vr1b[...])
  zj = mm(zj, vr2w[...]) + vr2b[...]
  x = relu((mm(hi, uv1a[...]) + mm(zj, uv1b[...]) + uv1bias[...]) * s3[...]
           + t3[...])
  x = relu((mm(x, uv2w[...]) + uv2b[...]) * s4[...] + t4[...])
  out_ref[...] = (jnp.sum(x * uv3wT[...], axis=1, keepdims=True) + uv3b[...])


def kernel(nodes_u, nodes_v, history_u, history_ur, history_v, history_vr,
           social_adj, embed_u, embed_i, embed_r,
           gu1_w, gu1_b, gu2_w, gu2_b, ai1_w, ai1_b, ai2_w, ai2_b, ai3_w,
           ai3_b, gv1_w, gv1_b, gv2_w, gv2_b, aI1_w, aI1_b, aI2_w, aI2_b,
           aI3_w, aI3_b, aS1_w, aS1_b, aS2_w, aS2_b, aS3_w, aS3_b, um1_w,
           um1_b, um2_w, um2_b, ur1_w, ur1_b, ur2_w, ur2_b, vr1_w, vr1_b,
           vr2_w, vr2_b, uv1_w, uv1_b, uv2_w, uv2_b, uv3_w, uv3_b,
           bn1_g, bn1_b, bn1_m, bn1_v, bn2_g, bn2_b, bn2_m, bn2_v,
           bn3_g, bn3_b, bn3_m, bn3_v, bn4_g, bn4_b, bn4_m, bn4_v):
  i32 = jnp.int32
  hist_v = history_v.astype(i32).reshape(B * L)
  soc = social_adj.astype(i32).reshape(B * L)
  hist_u = history_u.astype(i32).reshape(B * L)
  nu = nodes_u.astype(i32)
  nv = nodes_v.astype(i32)
  vr_ids = history_vr.astype(i32).reshape(B * L, 1)
  ur_ids = history_ur.astype(i32).reshape(B * L, 1)

  erp = jnp.pad(embed_r, ((0, 8 - NR), (0, 0)))

  def bn_fold(g, b, m, v):
    s = (g * lax.rsqrt(v + 1e-5)).reshape(1, -1)
    t = (b - m * g * lax.rsqrt(v + 1e-5)).reshape(1, -1)
    return s, t

  s1, t1 = bn_fold(bn1_g, bn1_b, bn1_m, bn1_v)
  s2, t2 = bn_fold(bn2_g, bn2_b, bn2_m, bn2_v)
  s3, t3 = bn_fold(bn3_g, bn3_b, bn3_m, bn3_v)
  s4, t4 = bn_fold(bn4_g, bn4_b, bn4_m, bn4_v)

  row = lambda b: b.reshape(1, -1)

  def bdiag(a, b):
    za = jnp.zeros_like(a)
    return jnp.concatenate(
        [jnp.concatenate([a, za], axis=1), jnp.concatenate([za, b], axis=1)],
        axis=0)

  weights = [
      bdiag(erp, erp),
      bdiag(gu1_w[:D], gv1_w[:D]), bdiag(gu1_w[D:], gv1_w[D:]),
      row(jnp.concatenate([gu1_b, gv1_b])),
      bdiag(gu2_w, gv2_w), row(jnp.concatenate([gu2_b, gv2_b])),
      bdiag(ai1_w[:D], aI1_w[:D]), bdiag(ai1_w[D:], aI1_w[D:]),
      row(jnp.concatenate([ai1_b, aI1_b])),
      bdiag(ai2_w, aI2_w), row(jnp.concatenate([ai2_b, aI2_b])),
      jnp.concatenate([ai3_w, aI3_w]).reshape(1, 2 * D),
      aS1_w[:D], aS1_w[D:], row(aS1_b), aS2_w, row(aS2_b), aS3_w.reshape(1, D),
      um1_w[:D], um1_w[D:], row(um1_b), um2_w, row(um2_b),
      ur1_w, row(ur1_b), ur2_w, row(ur2_b),
      vr1_w, row(vr1_b), vr2_w, row(vr2_b),
      uv1_w[:D], uv1_w[D:], row(uv1_b), uv2_w, row(uv2_b),
      uv3_w.reshape(1, 16), uv3_b.reshape(1, 1),
      s1, t1, s2, t2, s3, t3, s4, t4,
  ]

  big_spec = pl.BlockSpec((BB * L, D), lambda g: (g, 0))
  small_spec = pl.BlockSpec((BB, D), lambda g: (g, 0))
  ids_spec = pl.BlockSpec((BB * L, 1), lambda g: (g, 0))
  wspec = lambda a: pl.BlockSpec(a.shape, lambda g, _n=len(a.shape): (0,) * _n)

  in_specs = ([big_spec, big_spec, big_spec, small_spec, small_spec,
               ids_spec, ids_spec]
              + [wspec(w) for w in weights])

  tc_call = pl.pallas_call(
      _tc_body,
      grid=(G,),
      in_specs=in_specs,
      out_specs=pl.BlockSpec((BB, 1), lambda g: (g, 0)),
      out_shape=jax.ShapeDtypeStruct((CB, 1), _f32),
      compiler_params=pltpu.CompilerParams(
          dimension_semantics=("arbitrary",),
          vmem_limit_bytes=100 * 1024 * 1024),
  )

  outs = []
  for c in range(NCHUNK):
    lo, slo = c * BIG, c * CB
    pt, un, qa, piu, qj = _sc_gather(
        lax.slice(hist_v, (lo,), (lo + BIG,)),
        lax.slice(soc, (lo,), (lo + BIG,)),
        lax.slice(hist_u, (lo,), (lo + BIG,)),
        lax.slice(nu, (slo,), (slo + CB,)),
        lax.slice(nv, (slo,), (slo + CB,)),
        embed_u, embed_i)
    vr_c = lax.slice(vr_ids, (lo, 0), (lo + BIG, 1))
    ur_c = lax.slice(ur_ids, (lo, 0), (lo + BIG, 1))
    outs.append(tc_call(pt, qa, un, qj, piu, vr_c, ur_c, *weights))

  return jnp.concatenate(outs, axis=0)[:, 0]
